# baseline (speedup 1.0000x reference)
import functools

import jax
import jax.numpy as jnp
from jax import lax
from jax.experimental import pallas as pl
from jax.experimental.pallas import tpu as pltpu
from jax.experimental.pallas import tpu_sc as plsc

NS = 16
F32 = jnp.float32
BIG = 3.0e38
_HI = lax.Precision.HIGHEST


def _mm(a, b):
    return lax.dot_general(a, b, (((1,), (0,)), ((), ())),
                           preferred_element_type=F32, precision=_HI)


def _relu(v):
    return jnp.maximum(v, 0.0)


def _tables_body(x_ref, w1x_ref, w3_ref, b3_ref, xw_ref, xv_ref):
    xb = x_ref[...]
    xw_ref[...] = _mm(xb, w1x_ref[...])
    xv_ref[...] = _mm(xb, w3_ref[...]) + b3_ref[...]


def _tables(x, w1x, w3, b3r):
    n, cin = x.shape
    blk = 2000 if n % 2000 == 0 else n
    return pl.pallas_call(
        _tables_body,
        grid=(n // blk,),
        in_specs=[
            pl.BlockSpec((blk, cin), lambda i: (i, 0)),
            pl.BlockSpec(w1x.shape, lambda i: (0, 0)),
            pl.BlockSpec(w3.shape, lambda i: (0, 0)),
            pl.BlockSpec(b3r.shape, lambda i: (0, 0)),
        ],
        out_specs=[
            pl.BlockSpec((blk, w1x.shape[1]), lambda i: (i, 0)),
            pl.BlockSpec((blk, w3.shape[1]), lambda i: (i, 0)),
        ],
        out_shape=[
            jax.ShapeDtypeStruct((n, w1x.shape[1]), F32),
            jax.ShapeDtypeStruct((n, w3.shape[1]), F32),
        ],
    )(x, w1x, w3, b3r)


def _knn_body(n, p_ref, pt_ref, idx_ref, d2_ref):
    nt, _, ct = pt_ref.shape
    qb = p_ref.shape[0]
    qx = p_ref[:, 0:1]
    qy = p_ref[:, 1:2]
    qz = p_ref[:, 2:3]
    sqq = qx * qx + qy * qy + qz * qz
    qbf = p_ref[...].astype(jnp.bfloat16)
    coli = lax.broadcasted_iota(jnp.int32, (qb, ct), 1)
    big_i = jnp.int32(nt * ct)

    def d2tile(t, vmin):
        ptt = pt_ref[t]
        cx = ptt[0:1, :]
        cy = ptt[1:2, :]
        cz = ptt[2:3, :]
        sqc = cx * cx + cy * cy + cz * cz
        dot = lax.dot_general(qbf, ptt.astype(jnp.bfloat16),
                              (((1,), (0,)), ((), ())),
                              preferred_element_type=F32)
        col = coli + t * ct
        d2 = jnp.where(col < n, (sqq + sqc) - 2.0 * dot, BIG)
        d2_ref[t] = d2
        return jnp.minimum(vmin, jnp.min(d2, axis=1, keepdims=True))

    v0 = lax.fori_loop(0, nt, d2tile, jnp.full((qb, 1), BIG, F32))
    lane16 = lax.broadcasted_iota(jnp.int32, (qb, NS), 1)

    def kstep(k, carry):
        v, vprev, ciprev, idxacc = carry
        fresh = v != vprev

        def tstep(t, c):
            ci, vgt, cnt = c
            tile = d2_ref[t]
            col = coli + t * ct
            eq = tile == v
            valid = jnp.logical_and(
                eq, jnp.logical_or(fresh, col > ciprev))
            ci = jnp.minimum(
                ci, jnp.min(jnp.where(valid, col, big_i), axis=1,
                            keepdims=True))
            cnt = cnt + jnp.sum(valid.astype(jnp.int32), axis=1,
                                keepdims=True)
            vgt = jnp.minimum(
                vgt, jnp.min(jnp.where(tile > v, tile, BIG), axis=1,
                             keepdims=True))
            return (ci, vgt, cnt)

        ci, vgt, cnt = lax.fori_loop(
            0, nt, tstep,
            (jnp.full((qb, 1), big_i, jnp.int32),
             jnp.full((qb, 1), BIG, F32),
             jnp.zeros((qb, 1), jnp.int32)))
        vnext = jnp.where(cnt >= 2, v, vgt)
        idxacc = idxacc + ci * (lane16 == k).astype(jnp.int32)
        return (vnext, v, ci, idxacc)

    _, _, _, idxacc = lax.fori_loop(
        0, NS, kstep,
        (v0, jnp.full((qb, 1), BIG, F32),
         jnp.full((qb, 1), big_i, jnp.int32),
         jnp.zeros((qb, NS), jnp.int32)))
    idx_ref[...] = idxacc


def _knn(p, pt3):
    n = p.shape[0]
    nt, _, ct = pt3.shape
    qb = 400 if n % 400 == 0 else n
    return pl.pallas_call(
        functools.partial(_knn_body, n),
        grid=(n // qb,),
        in_specs=[
            pl.BlockSpec((qb, 3), lambda i: (i, 0)),
            pl.BlockSpec((nt, 3, ct), lambda i: (0, 0, 0)),
        ],
        out_specs=pl.BlockSpec((qb, NS), lambda i: (i, 0)),
        out_shape=jax.ShapeDtypeStruct((n, NS), jnp.int32),
        scratch_shapes=[pltpu.VMEM((nt, qb, ct), F32)],
    )(p, pt3)


def _sc_gather(idxf, xv, pxw):
    m = idxf.shape[0]
    dv = xv.shape[1]
    dp = pxw.shape[1]
    nw = 32
    rpw = m // nw
    ch = 200
    nch = rpw // ch
    mesh = plsc.VectorSubcoreMesh(core_axis_name="c", subcore_axis_name="s")

    @functools.partial(
        pl.kernel, mesh=mesh,
        out_type=[jax.ShapeDtypeStruct((m, dv), F32),
                  jax.ShapeDtypeStruct((m, dp), F32)],
        scratch_types=[
            pltpu.VMEM((ch,), jnp.int32),
            pltpu.VMEM((ch, dv), F32),
            pltpu.VMEM((ch, dp), F32),
            pltpu.SemaphoreType.DMA,
            pltpu.SemaphoreType.DMA,
        ],
    )
    def gk(idx_hbm, xv_hbm, pxw_hbm, outv_hbm, outp_hbm,
           idx_v, rows_v, rows2_v, sem1, sem2):
        wid = lax.axis_index("s") * 2 + lax.axis_index("c")
        base = wid * rpw

        def body(i, carry):
            off = base + i * ch
            pltpu.sync_copy(idx_hbm.at[pl.ds(off, ch)], idx_v)
            c1 = pltpu.async_copy(xv_hbm.at[idx_v], rows_v, sem1)
            c2 = pltpu.async_copy(pxw_hbm.at[idx_v], rows2_v, sem2)
            c1.wait()
            c2.wait()
            pltpu.sync_copy(rows_v, outv_hbm.at[pl.ds(off, ch)])
            pltpu.sync_copy(rows2_v, outp_hbm.at[pl.ds(off, ch)])
            return carry

        lax.fori_loop(0, nch, body, 0)

    return gk(idxf, xv, pxw)


def _pemom_body(g32_ref, prep_ref, wp1_ref, bp1_ref, m1_ref, m2_ref):
    pr = g32_ref[:, 0:3] - prep_ref[...]
    pe = _mm(pr, wp1_ref[...]) + bp1_ref[...]

    @pl.when(pl.program_id(0) == 0)
    def _():
        m1_ref[...] = jnp.zeros_like(m1_ref)
        m2_ref[...] = jnp.zeros_like(m2_ref)

    m1_ref[...] += jnp.sum(pe, axis=0, keepdims=True)
    m2_ref[...] += jnp.sum(pe * pe, axis=0, keepdims=True)


def _pemom(g32, prep, wp1, bp1r):
    m = g32.shape[0]
    rb = 8000 if m % 8000 == 0 else m
    return pl.pallas_call(
        _pemom_body,
        grid=(m // rb,),
        in_specs=[
            pl.BlockSpec((rb, g32.shape[1]), lambda i: (i, 0)),
            pl.BlockSpec((rb, 3), lambda i: (i, 0)),
            pl.BlockSpec((3, 3), lambda i: (0, 0)),
            pl.BlockSpec((1, 3), lambda i: (0, 0)),
        ],
        out_specs=[
            pl.BlockSpec((1, 3), lambda i: (0, 0)),
            pl.BlockSpec((1, 3), lambda i: (0, 0)),
        ],
        out_shape=[jax.ShapeDtypeStruct((1, 3), F32),
                   jax.ShapeDtypeStruct((1, 3), F32)],
    )(g32, prep, wp1, bp1r)


def _pass1_body(g32_ref, prep_ref, w1p_ref, b1_ref, tt_ref, tr_ref, bw2_ref,
                bb_ref, wp1f_ref, cp1f_ref, wp2s_ref, bp2s_ref, w2a_ref,
                e2_ref, pr_ref, m1_ref, m2_ref):
    pr = g32_ref[:, 0:3] - prep_ref[...]
    pr_ref[...] = pr
    e = _relu(_mm(pr, w1p_ref[...]) + g32_ref[:, 16:32] + b1_ref[...])
    e2sq = _mm(e, tt_ref[...]) * _mm(e, tr_ref[...])
    eb = _mm(e2sq, bw2_ref[...]) + bb_ref[...]
    peb = _relu(_mm(pr, wp1f_ref[...]) + cp1f_ref[...])
    sh = _mm(peb, wp2s_ref[...]) + bp2s_ref[...]
    e2 = jnp.concatenate([eb, sh], axis=1)
    e2_ref[...] = e2
    h1 = _mm(e2, w2a_ref[...])

    @pl.when(pl.program_id(0) == 0)
    def _():
        m1_ref[...] = jnp.zeros_like(m1_ref)
        m2_ref[...] = jnp.zeros_like(m2_ref)

    m1_ref[...] += jnp.sum(h1, axis=0, keepdims=True)
    m2_ref[...] += jnp.sum(h1 * h1, axis=0, keepdims=True)


def _pass1(g32, prep, w1p, b1r, tt, tr, bw2, bbr, wp1f, cp1fr, wp2s, bp2sr,
           w2a):
    m = g32.shape[0]
    rb = 4000 if m % 4000 == 0 else m
    mid = w2a.shape[1]
    full = lambda a: pl.BlockSpec(a.shape, lambda i: (0, 0))
    return pl.pallas_call(
        _pass1_body,
        grid=(m // rb,),
        in_specs=[
            pl.BlockSpec((rb, g32.shape[1]), lambda i: (i, 0)),
            pl.BlockSpec((rb, 3), lambda i: (i, 0)),
            full(w1p), full(b1r), full(tt), full(tr), full(bw2), full(bbr),
            full(wp1f), full(cp1fr), full(wp2s), full(bp2sr), full(w2a),
        ],
        out_specs=[
            pl.BlockSpec((rb, 2 * NS), lambda i: (i, 0)),
            pl.BlockSpec((rb, 3), lambda i: (i, 0)),
            pl.BlockSpec((1, mid), lambda i: (0, 0)),
            pl.BlockSpec((1, mid), lambda i: (0, 0)),
        ],
        out_shape=[
            jax.ShapeDtypeStruct((m, 2 * NS), F32),
            jax.ShapeDtypeStruct((m, 3), F32),
            jax.ShapeDtypeStruct((1, mid), F32),
            jax.ShapeDtypeStruct((1, mid), F32),
        ],
    )(g32, prep, w1p, b1r, tt, tr, bw2, bbr, wp1f, cp1fr, wp2s, bp2sr, w2a)


def _pass2_body(e2_ref, w2af_ref, c2af_ref, w2b_ref, m1_ref, m2_ref):
    h2 = _relu(_mm(e2_ref[...], w2af_ref[...]) + c2af_ref[...])
    hb = _mm(h2, w2b_ref[...])

    @pl.when(pl.program_id(0) == 0)
    def _():
        m1_ref[...] = jnp.zeros_like(m1_ref)
        m2_ref[...] = jnp.zeros_like(m2_ref)

    m1_ref[...] += jnp.sum(hb, axis=0, keepdims=True)
    m2_ref[...] += jnp.sum(hb * hb, axis=0, keepdims=True)


def _pass2(e2, w2af, c2afr, w2b):
    m = e2.shape[0]
    rb = 8000 if m % 8000 == 0 else m
    k = w2b.shape[1]
    full = lambda a: pl.BlockSpec(a.shape, lambda i: (0, 0))
    return pl.pallas_call(
        _pass2_body,
        grid=(m // rb,),
        in_specs=[
            pl.BlockSpec((rb, e2.shape[1]), lambda i: (i, 0)),
            full(w2af), full(c2afr), full(w2b),
        ],
        out_specs=[
            pl.BlockSpec((1, k), lambda i: (0, 0)),
            pl.BlockSpec((1, k), lambda i: (0, 0)),
        ],
        out_shape=[jax.ShapeDtypeStruct((1, k), F32),
                   jax.ShapeDtypeStruct((1, k), F32)],
    )(e2, w2af, c2afr, w2b)


def _pass3_body(bp, e2_ref, pr_ref, xvg_ref, w2af_ref, c2af_ref, w2bf_ref,
                c2bf_ref, w2c_ref, b2c_ref, wp1f_ref, cp1f_ref, wp2_ref,
                bp2_ref, tt128_ref, xk_ref, out_ref):
    h2 = _relu(_mm(e2_ref[...], w2af_ref[...]) + c2af_ref[...])
    h3 = _relu(_mm(h2, w2bf_ref[...]) + c2bf_ref[...])
    en = _mm(h3, w2c_ref[...]) + b2c_ref[...]
    k = en.shape[1]
    en3 = en.reshape(bp, NS, k)
    mx = jnp.max(en3, axis=1, keepdims=True)
    ez = jnp.exp(en3 - mx)
    sm = ez / jnp.sum(ez, axis=1, keepdims=True)
    wsm = sm.reshape(bp * NS, k)
    peb = _relu(_mm(pr_ref[...], wp1f_ref[...]) + cp1f_ref[...])
    pemb = _mm(peb, wp2_ref[...]) + bp2_ref[...]
    xk = (xvg_ref[...] + pemb) * _mm(wsm, tt128_ref[...])
    xk_ref[...] = xk
    out_ref[...] = jnp.sum(xk.reshape(bp, NS, xk.shape[1]), axis=1)


def _pass3(e2, pr, xvg, w2af, c2afr, w2bf, c2bfr, w2c, b2cr, wp1f, cp1fr,
           wp2, bp2r, tt128):
    m = e2.shape[0]
    n = m // NS
    bp = 400 if n % 400 == 0 else n
    rb = bp * NS
    dout = wp2.shape[1]
    full = lambda a: pl.BlockSpec(a.shape, lambda i: (0, 0))
    return pl.pallas_call(
        functools.partial(_pass3_body, bp),
        grid=(m // rb,),
        in_specs=[
            pl.BlockSpec((rb, e2.shape[1]), lambda i: (i, 0)),
            pl.BlockSpec((rb, 3), lambda i: (i, 0)),
            pl.BlockSpec((rb, dout), lambda i: (i, 0)),
            full(w2af), full(c2afr), full(w2bf), full(c2bfr), full(w2c),
            full(b2cr), full(wp1f), full(cp1fr), full(wp2), full(bp2r),
            full(tt128),
        ],
        out_specs=[
            pl.BlockSpec((rb, dout), lambda i: (i, 0)),
            pl.BlockSpec((bp, dout), lambda i: (i, 0)),
        ],
        out_shape=[
            jax.ShapeDtypeStruct((m, dout), F32),
            jax.ShapeDtypeStruct((n, dout), F32),
        ],
    )(e2, pr, xvg, w2af, c2afr, w2bf, c2bfr, w2c, b2cr, wp1f, cp1fr, wp2,
      bp2r, tt128)


def kernel(p, x, o, w1, b1, bw, bb, wp1, bp1, gp, betap, wp2, bp2,
           w2a, g2a, b2a, w2b, g2b, b2b, w2c, b2c, w3, b3):
    n = p.shape[0]
    out_c = wp2.shape[1]
    share = out_c // w2c.shape[1]
    m = n * NS
    eps = 1e-5

    xw, xv = _tables(x, w1[3:], w3, b3.reshape(1, -1))

    ct = 512
    npad = ((n + ct - 1) // ct) * ct
    pt3 = (jnp.zeros((3, npad), F32).at[:, :n].set(p.T)
           .reshape(3, npad // ct, ct).transpose(1, 0, 2))
    knn_idx = _knn(p, pt3)

    pxw = jnp.concatenate(
        [p, jnp.zeros((n, 13), F32), xw, jnp.zeros((n, 96), F32)], axis=1)
    idxf = knn_idx.reshape(-1)
    xvg, g32 = _sc_gather(idxf, xv, pxw)

    prep = jnp.repeat(p, NS, axis=0)

    pm1, pm2 = _pemom(g32, prep, wp1, bp1.reshape(1, -1))
    mean_pe = pm1[0] / m
    var_pe = pm2[0] / m - mean_pe * mean_pe
    spe = gp / jnp.sqrt(var_pe + eps)
    wp1f = wp1 * spe[None, :]
    cp1f = ((bp1 - mean_pe) * spe + betap).reshape(1, -1)

    wp2s = wp2.reshape(3, share, out_c // share).sum(axis=1)
    bp2s = bp2.reshape(share, out_c // share).sum(axis=0).reshape(1, -1)

    ar = jnp.arange(NS * NS, dtype=jnp.int32)
    ai = jnp.arange(NS, dtype=jnp.int32)
    tt = (ar[None, :] % NS == ai[:, None]).astype(F32)
    tr = (ar[None, :] // NS == ai[:, None]).astype(F32)
    bw2 = bw.transpose(1, 2, 0).reshape(NS * NS, NS)

    e2, pr, am1, am2 = _pass1(
        g32, prep, w1[:3], b1.reshape(1, -1), tt, tr, bw2, bb.reshape(1, -1),
        wp1f, cp1f, wp2s, bp2s, w2a)

    mu1 = am1[0] / m
    var1 = am2[0] / m - mu1 * mu1
    sa = g2a / jnp.sqrt(var1 + eps)
    w2af = w2a * sa[None, :]
    c2af = (b2a - mu1 * sa).reshape(1, -1)

    bm1, bm2 = _pass2(e2, w2af, c2af, w2b)
    mu2 = bm1[0] / m
    var2 = bm2[0] / m - mu2 * mu2
    sb = g2b / jnp.sqrt(var2 + eps)
    w2bf = w2b * sb[None, :]
    c2bf = (b2b - mu2 * sb).reshape(1, -1)

    arl = jnp.arange(out_c, dtype=jnp.int32)
    tt128 = (arl[None, :] % (out_c // share) == ai[:, None]).astype(F32)

    xk, out = _pass3(e2, pr, xvg, w2af, c2af, w2bf, c2bf, w2c,
                     b2c.reshape(1, -1), wp1f, cp1f, wp2, bp2.reshape(1, -1),
                     tt128)

    return (out, xk.reshape(n, NS, out_c), knn_idx, pr.reshape(n, NS, 3))


# speedup vs baseline: 1.9067x; 1.9067x over previous
import functools

import jax
import jax.numpy as jnp
from jax import lax
from jax.experimental import pallas as pl
from jax.experimental.pallas import tpu as pltpu
from jax.experimental.pallas import tpu_sc as plsc

NS = 16
F32 = jnp.float32
BIG = 3.0e38
_HI = lax.Precision.HIGHEST


def _mm(a, b):
    return lax.dot_general(a, b, (((1,), (0,)), ((), ())),
                           preferred_element_type=F32, precision=_HI)


def _relu(v):
    return jnp.maximum(v, 0.0)


def _tables_body(x_ref, w1x_ref, w3_ref, b3_ref, xw_ref, xv_ref):
    xb = x_ref[...]
    xw_ref[...] = _mm(xb, w1x_ref[...])
    xv_ref[...] = _mm(xb, w3_ref[...]) + b3_ref[...]


def _tables(x, w1x, w3, b3r):
    n, cin = x.shape
    blk = 2000 if n % 2000 == 0 else n
    return pl.pallas_call(
        _tables_body,
        grid=(n // blk,),
        in_specs=[
            pl.BlockSpec((blk, cin), lambda i: (i, 0)),
            pl.BlockSpec(w1x.shape, lambda i: (0, 0)),
            pl.BlockSpec(w3.shape, lambda i: (0, 0)),
            pl.BlockSpec(b3r.shape, lambda i: (0, 0)),
        ],
        out_specs=[
            pl.BlockSpec((blk, w1x.shape[1]), lambda i: (i, 0)),
            pl.BlockSpec((blk, w3.shape[1]), lambda i: (i, 0)),
        ],
        out_shape=[
            jax.ShapeDtypeStruct((n, w1x.shape[1]), F32),
            jax.ShapeDtypeStruct((n, w3.shape[1]), F32),
        ],
    )(x, w1x, w3, b3r)


def _knn_body(n, p_ref, pt_ref, idx_ref, d2_ref):
    nt, _, ct = pt_ref.shape
    qb = p_ref.shape[0]
    qx = p_ref[:, 0:1]
    qy = p_ref[:, 1:2]
    qz = p_ref[:, 2:3]
    sqq = qx * qx + qy * qy + qz * qz
    qbf = p_ref[...].astype(jnp.bfloat16)
    coli = lax.broadcasted_iota(jnp.int32, (qb, ct), 1)
    big_i = jnp.int32(nt * ct)

    def d2tile(t, vmin):
        ptt = pt_ref[t]
        cx = ptt[0:1, :]
        cy = ptt[1:2, :]
        cz = ptt[2:3, :]
        sqc = cx * cx + cy * cy + cz * cz
        dot = lax.dot_general(qbf, ptt.astype(jnp.bfloat16),
                              (((1,), (0,)), ((), ())),
                              preferred_element_type=F32)
        col = coli + t * ct
        d2 = jnp.where(col < n, (sqq + sqc) - 2.0 * dot, BIG)
        d2_ref[t] = d2
        return jnp.minimum(vmin, jnp.min(d2, axis=1, keepdims=True))

    lax.fori_loop(0, nt, d2tile, jnp.full((qb, 1), BIG, F32))
    lane16 = lax.broadcasted_iota(jnp.int32, (qb, NS), 1)

    def kstep(k, carry):
        ci_prev, idxacc = carry

        def sweep_v(t, vm):
            tile = d2_ref[t]
            col = coli + t * ct
            tile = jnp.where(col == ci_prev, BIG, tile)
            d2_ref[t] = tile
            return jnp.minimum(vm, jnp.min(tile, axis=1, keepdims=True))

        v = lax.fori_loop(0, nt, sweep_v, jnp.full((qb, 1), BIG, F32))

        def sweep_i(t, ci):
            tile = d2_ref[t]
            col = coli + t * ct
            return jnp.minimum(
                ci,
                jnp.min(jnp.where(tile == v, col, big_i), axis=1,
                        keepdims=True))

        ci = lax.fori_loop(0, nt, sweep_i,
                           jnp.full((qb, 1), big_i, jnp.int32))
        idxacc = idxacc + ci * (lane16 == k).astype(jnp.int32)
        return (ci, idxacc)

    _, idxacc = lax.fori_loop(
        0, NS, kstep,
        (jnp.full((qb, 1), big_i, jnp.int32),
         jnp.zeros((qb, NS), jnp.int32)))
    idx_ref[...] = idxacc


def _knn(p, pt3):
    n = p.shape[0]
    nt, _, ct = pt3.shape
    qb = 400 if n % 400 == 0 else n
    return pl.pallas_call(
        functools.partial(_knn_body, n),
        grid=(n // qb,),
        in_specs=[
            pl.BlockSpec((qb, 3), lambda i: (i, 0)),
            pl.BlockSpec((nt, 3, ct), lambda i: (0, 0, 0)),
        ],
        out_specs=pl.BlockSpec((qb, NS), lambda i: (i, 0)),
        out_shape=jax.ShapeDtypeStruct((n, NS), jnp.int32),
        scratch_shapes=[pltpu.VMEM((nt, qb, ct), F32)],
    )(p, pt3)


def _sc_gather(idxf, xv, pxw):
    m = idxf.shape[0]
    dv = xv.shape[1]
    dp = pxw.shape[1]
    nw = 32
    rpw = m // nw
    ch = 200
    nch = rpw // ch
    mesh = plsc.VectorSubcoreMesh(core_axis_name="c", subcore_axis_name="s")

    @functools.partial(
        pl.kernel, mesh=mesh,
        out_type=[jax.ShapeDtypeStruct((m, dv), F32),
                  jax.ShapeDtypeStruct((m, dp), F32)],
        scratch_types=[
            pltpu.VMEM((ch,), jnp.int32),
            pltpu.VMEM((ch, dv), F32),
            pltpu.VMEM((ch, dp), F32),
            pltpu.SemaphoreType.DMA,
            pltpu.SemaphoreType.DMA,
        ],
    )
    def gk(idx_hbm, xv_hbm, pxw_hbm, outv_hbm, outp_hbm,
           idx_v, rows_v, rows2_v, sem1, sem2):
        wid = lax.axis_index("s") * 2 + lax.axis_index("c")
        base = wid * rpw

        def body(i, carry):
            off = base + i * ch
            pltpu.sync_copy(idx_hbm.at[pl.ds(off, ch)], idx_v)
            c1 = pltpu.async_copy(xv_hbm.at[idx_v], rows_v, sem1)
            c2 = pltpu.async_copy(pxw_hbm.at[idx_v], rows2_v, sem2)
            c1.wait()
            c2.wait()
            pltpu.sync_copy(rows_v, outv_hbm.at[pl.ds(off, ch)])
            pltpu.sync_copy(rows2_v, outp_hbm.at[pl.ds(off, ch)])
            return carry

        lax.fori_loop(0, nch, body, 0)

    return gk(idxf, xv, pxw)


def _pemom_body(g32_ref, prep_ref, wp1_ref, bp1_ref, m1_ref, m2_ref):
    pr = g32_ref[:, 0:3] - prep_ref[...]
    pe = _mm(pr, wp1_ref[...]) + bp1_ref[...]

    @pl.when(pl.program_id(0) == 0)
    def _():
        m1_ref[...] = jnp.zeros_like(m1_ref)
        m2_ref[...] = jnp.zeros_like(m2_ref)

    m1_ref[...] += jnp.sum(pe, axis=0, keepdims=True)
    m2_ref[...] += jnp.sum(pe * pe, axis=0, keepdims=True)


def _pemom(g32, prep, wp1, bp1r):
    m = g32.shape[0]
    rb = 8000 if m % 8000 == 0 else m
    return pl.pallas_call(
        _pemom_body,
        grid=(m // rb,),
        in_specs=[
            pl.BlockSpec((rb, g32.shape[1]), lambda i: (i, 0)),
            pl.BlockSpec((rb, 3), lambda i: (i, 0)),
            pl.BlockSpec((3, 3), lambda i: (0, 0)),
            pl.BlockSpec((1, 3), lambda i: (0, 0)),
        ],
        out_specs=[
            pl.BlockSpec((1, 3), lambda i: (0, 0)),
            pl.BlockSpec((1, 3), lambda i: (0, 0)),
        ],
        out_shape=[jax.ShapeDtypeStruct((1, 3), F32),
                   jax.ShapeDtypeStruct((1, 3), F32)],
    )(g32, prep, wp1, bp1r)


def _pass1_body(g32_ref, prep_ref, w1p_ref, b1_ref, tt_ref, tr_ref, bw2_ref,
                bb_ref, wp1f_ref, cp1f_ref, wp2s_ref, bp2s_ref, w2a_ref,
                e2_ref, pr_ref, m1_ref, m2_ref):
    pr = g32_ref[:, 0:3] - prep_ref[...]
    pr_ref[...] = pr
    e = _relu(_mm(pr, w1p_ref[...]) + g32_ref[:, 16:32] + b1_ref[...])
    e2sq = _mm(e, tt_ref[...]) * _mm(e, tr_ref[...])
    eb = _mm(e2sq, bw2_ref[...]) + bb_ref[...]
    peb = _relu(_mm(pr, wp1f_ref[...]) + cp1f_ref[...])
    sh = _mm(peb, wp2s_ref[...]) + bp2s_ref[...]
    e2 = jnp.concatenate([eb, sh], axis=1)
    e2_ref[...] = e2
    h1 = _mm(e2, w2a_ref[...])

    @pl.when(pl.program_id(0) == 0)
    def _():
        m1_ref[...] = jnp.zeros_like(m1_ref)
        m2_ref[...] = jnp.zeros_like(m2_ref)

    m1_ref[...] += jnp.sum(h1, axis=0, keepdims=True)
    m2_ref[...] += jnp.sum(h1 * h1, axis=0, keepdims=True)


def _pass1(g32, prep, w1p, b1r, tt, tr, bw2, bbr, wp1f, cp1fr, wp2s, bp2sr,
           w2a):
    m = g32.shape[0]
    rb = 4000 if m % 4000 == 0 else m
    mid = w2a.shape[1]
    full = lambda a: pl.BlockSpec(a.shape, lambda i: (0, 0))
    return pl.pallas_call(
        _pass1_body,
        grid=(m // rb,),
        in_specs=[
            pl.BlockSpec((rb, g32.shape[1]), lambda i: (i, 0)),
            pl.BlockSpec((rb, 3), lambda i: (i, 0)),
            full(w1p), full(b1r), full(tt), full(tr), full(bw2), full(bbr),
            full(wp1f), full(cp1fr), full(wp2s), full(bp2sr), full(w2a),
        ],
        out_specs=[
            pl.BlockSpec((rb, 2 * NS), lambda i: (i, 0)),
            pl.BlockSpec((rb, 3), lambda i: (i, 0)),
            pl.BlockSpec((1, mid), lambda i: (0, 0)),
            pl.BlockSpec((1, mid), lambda i: (0, 0)),
        ],
        out_shape=[
            jax.ShapeDtypeStruct((m, 2 * NS), F32),
            jax.ShapeDtypeStruct((m, 3), F32),
            jax.ShapeDtypeStruct((1, mid), F32),
            jax.ShapeDtypeStruct((1, mid), F32),
        ],
    )(g32, prep, w1p, b1r, tt, tr, bw2, bbr, wp1f, cp1fr, wp2s, bp2sr, w2a)


def _pass2_body(e2_ref, w2af_ref, c2af_ref, w2b_ref, m1_ref, m2_ref):
    h2 = _relu(_mm(e2_ref[...], w2af_ref[...]) + c2af_ref[...])
    hb = _mm(h2, w2b_ref[...])

    @pl.when(pl.program_id(0) == 0)
    def _():
        m1_ref[...] = jnp.zeros_like(m1_ref)
        m2_ref[...] = jnp.zeros_like(m2_ref)

    m1_ref[...] += jnp.sum(hb, axis=0, keepdims=True)
    m2_ref[...] += jnp.sum(hb * hb, axis=0, keepdims=True)


def _pass2(e2, w2af, c2afr, w2b):
    m = e2.shape[0]
    rb = 8000 if m % 8000 == 0 else m
    k = w2b.shape[1]
    full = lambda a: pl.BlockSpec(a.shape, lambda i: (0, 0))
    return pl.pallas_call(
        _pass2_body,
        grid=(m // rb,),
        in_specs=[
            pl.BlockSpec((rb, e2.shape[1]), lambda i: (i, 0)),
            full(w2af), full(c2afr), full(w2b),
        ],
        out_specs=[
            pl.BlockSpec((1, k), lambda i: (0, 0)),
            pl.BlockSpec((1, k), lambda i: (0, 0)),
        ],
        out_shape=[jax.ShapeDtypeStruct((1, k), F32),
                   jax.ShapeDtypeStruct((1, k), F32)],
    )(e2, w2af, c2afr, w2b)


def _pass3_body(bp, e2_ref, pr_ref, xvg_ref, w2af_ref, c2af_ref, w2bf_ref,
                c2bf_ref, w2c_ref, b2c_ref, wp1f_ref, cp1f_ref, wp2_ref,
                bp2_ref, tt128_ref, xk_ref, out_ref):
    h2 = _relu(_mm(e2_ref[...], w2af_ref[...]) + c2af_ref[...])
    h3 = _relu(_mm(h2, w2bf_ref[...]) + c2bf_ref[...])
    en = _mm(h3, w2c_ref[...]) + b2c_ref[...]
    k = en.shape[1]
    en3 = en.reshape(bp, NS, k)
    mx = jnp.max(en3, axis=1, keepdims=True)
    ez = jnp.exp(en3 - mx)
    sm = ez / jnp.sum(ez, axis=1, keepdims=True)
    wsm = sm.reshape(bp * NS, k)
    peb = _relu(_mm(pr_ref[...], wp1f_ref[...]) + cp1f_ref[...])
    pemb = _mm(peb, wp2_ref[...]) + bp2_ref[...]
    xk = (xvg_ref[...] + pemb) * _mm(wsm, tt128_ref[...])
    xk_ref[...] = xk
    out_ref[...] = jnp.sum(xk.reshape(bp, NS, xk.shape[1]), axis=1)


def _pass3(e2, pr, xvg, w2af, c2afr, w2bf, c2bfr, w2c, b2cr, wp1f, cp1fr,
           wp2, bp2r, tt128):
    m = e2.shape[0]
    n = m // NS
    bp = 400 if n % 400 == 0 else n
    rb = bp * NS
    dout = wp2.shape[1]
    full = lambda a: pl.BlockSpec(a.shape, lambda i: (0, 0))
    return pl.pallas_call(
        functools.partial(_pass3_body, bp),
        grid=(m // rb,),
        in_specs=[
            pl.BlockSpec((rb, e2.shape[1]), lambda i: (i, 0)),
            pl.BlockSpec((rb, 3), lambda i: (i, 0)),
            pl.BlockSpec((rb, dout), lambda i: (i, 0)),
            full(w2af), full(c2afr), full(w2bf), full(c2bfr), full(w2c),
            full(b2cr), full(wp1f), full(cp1fr), full(wp2), full(bp2r),
            full(tt128),
        ],
        out_specs=[
            pl.BlockSpec((rb, dout), lambda i: (i, 0)),
            pl.BlockSpec((bp, dout), lambda i: (i, 0)),
        ],
        out_shape=[
            jax.ShapeDtypeStruct((m, dout), F32),
            jax.ShapeDtypeStruct((n, dout), F32),
        ],
    )(e2, pr, xvg, w2af, c2afr, w2bf, c2bfr, w2c, b2cr, wp1f, cp1fr, wp2,
      bp2r, tt128)


def kernel(p, x, o, w1, b1, bw, bb, wp1, bp1, gp, betap, wp2, bp2,
           w2a, g2a, b2a, w2b, g2b, b2b, w2c, b2c, w3, b3):
    n = p.shape[0]
    out_c = wp2.shape[1]
    share = out_c // w2c.shape[1]
    m = n * NS
    eps = 1e-5

    xw, xv = _tables(x, w1[3:], w3, b3.reshape(1, -1))

    ct = 2048
    npad = ((n + ct - 1) // ct) * ct
    pt3 = (jnp.zeros((3, npad), F32).at[:, :n].set(p.T)
           .reshape(3, npad // ct, ct).transpose(1, 0, 2))
    knn_idx = _knn(p, pt3)

    pxw = jnp.concatenate(
        [p, jnp.zeros((n, 13), F32), xw, jnp.zeros((n, 96), F32)], axis=1)
    idxf = knn_idx.reshape(-1)
    xvg, g32 = _sc_gather(idxf, xv, pxw)

    prep = jnp.repeat(p, NS, axis=0)

    pm1, pm2 = _pemom(g32, prep, wp1, bp1.reshape(1, -1))
    mean_pe = pm1[0] / m
    var_pe = pm2[0] / m - mean_pe * mean_pe
    spe = gp / jnp.sqrt(var_pe + eps)
    wp1f = wp1 * spe[None, :]
    cp1f = ((bp1 - mean_pe) * spe + betap).reshape(1, -1)

    wp2s = wp2.reshape(3, share, out_c // share).sum(axis=1)
    bp2s = bp2.reshape(share, out_c // share).sum(axis=0).reshape(1, -1)

    ar = jnp.arange(NS * NS, dtype=jnp.int32)
    ai = jnp.arange(NS, dtype=jnp.int32)
    tt = (ar[None, :] % NS == ai[:, None]).astype(F32)
    tr = (ar[None, :] // NS == ai[:, None]).astype(F32)
    bw2 = bw.transpose(1, 2, 0).reshape(NS * NS, NS)

    e2, pr, am1, am2 = _pass1(
        g32, prep, w1[:3], b1.reshape(1, -1), tt, tr, bw2, bb.reshape(1, -1),
        wp1f, cp1f, wp2s, bp2s, w2a)

    mu1 = am1[0] / m
    var1 = am2[0] / m - mu1 * mu1
    sa = g2a / jnp.sqrt(var1 + eps)
    w2af = w2a * sa[None, :]
    c2af = (b2a - mu1 * sa).reshape(1, -1)

    bm1, bm2 = _pass2(e2, w2af, c2af, w2b)
    mu2 = bm1[0] / m
    var2 = bm2[0] / m - mu2 * mu2
    sb = g2b / jnp.sqrt(var2 + eps)
    w2bf = w2b * sb[None, :]
    c2bf = (b2b - mu2 * sb).reshape(1, -1)

    arl = jnp.arange(out_c, dtype=jnp.int32)
    tt128 = (arl[None, :] % (out_c // share) == ai[:, None]).astype(F32)

    xk, out = _pass3(e2, pr, xvg, w2af, c2af, w2bf, c2bf, w2c,
                     b2c.reshape(1, -1), wp1f, cp1f, wp2, bp2.reshape(1, -1),
                     tt128)

    return (out, xk.reshape(n, NS, out_c), knn_idx, pr.reshape(n, NS, 3))
